# Initial kernel scaffold; baseline (speedup 1.0000x reference)
import jax
import jax.numpy as jnp
from jax.experimental import pallas as pl


def kernel(x, h, g, neighbor_index, neighbor_mask, Wh, Wn, U, u_w, u_b, V_w, V_b):
    raise NotImplementedError("write your pallas kernel here")


# speedup vs baseline: 10.9407x; 10.9407x over previous
import functools

import jax
import jax.numpy as jnp
from jax import lax
from jax.experimental import pallas as pl
from jax.experimental.pallas import tpu as pltpu
from jax.experimental.pallas import tpu_sc as plsc

_B, _S, _K, _H = 16, 2048, 16, 128
_NC, _NS, _L = 2, 16, 16
_NW = _NC * _NS
_ROWS = _B * _S
_SS = _ROWS // _NW
_CH = 128
_NCHUNK = _SS // _CH


def _gather_body(newh_hbm, ni_hbm, mask_hbm, out_hbm, ni_v, mask_v,
                 acc_v, sem):
    wid = lax.axis_index("s") * _NC + lax.axis_index("c")
    row0 = wid * _SS
    b = row0 // _S
    boff = b * (_S + 1)
    scol0 = row0 % _S

    def chunk_body(c, carry):
        scol = scol0 + c * _CH
        pltpu.sync_copy(ni_hbm.at[pl.ds(b * _K, _K), pl.ds(scol, _CH)], ni_v)
        pltpu.sync_copy(mask_hbm.at[pl.ds(b * _K, _K), pl.ds(scol, _CH)],
                        mask_v)

        def vec_body(j, carry2):
            k = j // (_CH // _L)
            off = (j % (_CH // _L)) * _L
            sl = pl.ds(off, _L)
            ni_v[k, sl] = ni_v[k, sl] * mask_v[k, sl] + boff
            return carry2

        lax.fori_loop(0, _K * _CH // _L, vec_body, 0)

        first = pltpu.async_copy(newh_hbm.at[ni_v.at[0]], acc_v, sem)
        first.wait()
        rest = [
            pltpu.async_copy(newh_hbm.at[ni_v.at[k]], acc_v, sem, add=True)
            for k in range(1, _K)
        ]
        for d in rest:
            d.wait()
        pltpu.sync_copy(acc_v, out_hbm.at[pl.ds(row0 + c * _CH, _CH), :])
        return carry

    lax.fori_loop(0, _NCHUNK, chunk_body, 0)


_gather_call = functools.partial(
    pl.kernel,
    out_type=jax.ShapeDtypeStruct((_ROWS, _H), jnp.float32),
    mesh=plsc.VectorSubcoreMesh(core_axis_name="c", subcore_axis_name="s"),
    scratch_types=[
        pltpu.VMEM((_K, _CH), jnp.int32),
        pltpu.VMEM((_K, _CH), jnp.int32),
        pltpu.VMEM((_CH, _H), jnp.float32),
        pltpu.SemaphoreType.DMA,
    ],
)(_gather_body)


def _mm_body(a_ref, w_ref, o_ref):
    o_ref[...] = 0.0625 * lax.dot_general(
        a_ref[...], w_ref[...], (((1,), (1,)), ((), ())),
        preferred_element_type=jnp.float32,
        precision=lax.Precision.HIGHEST)


_BM = 512
_mm_call = pl.pallas_call(
    _mm_body,
    grid=(_ROWS // _BM,),
    in_specs=[
        pl.BlockSpec((_BM, _H), lambda i: (i, 0)),
        pl.BlockSpec((_H, _H), lambda i: (0, 0)),
    ],
    out_specs=pl.BlockSpec((_BM, _H), lambda i: (i, 0)),
    out_shape=jax.ShapeDtypeStruct((_ROWS, _H), jnp.float32),
)


def kernel(x, h, g, neighbor_index, neighbor_mask, Wh, Wn, U, u_w, u_b,
           V_w, V_b):
    newh = jnp.concatenate(
        [jnp.zeros((_B, 1, _H), dtype=h.dtype), h], axis=1
    ).reshape(_B * (_S + 1), _H)
    ni = neighbor_index.transpose(0, 2, 1).reshape(_B * _K, _S)
    mask = neighbor_mask.transpose(0, 2, 1).reshape(_B * _K, _S)
    acc = _gather_call(newh, ni, mask)
    out = _mm_call(acc, Wn)
    return out.reshape(_B, _S, _H)
